# baseline (speedup 1.0000x reference)
import functools

import jax
import jax.numpy as jnp
from jax import lax
from jax.experimental import pallas as pl
from jax.experimental.pallas import tpu as pltpu
from jax.experimental.pallas import tpu_sc as plsc

NUM_TOKENS = 16384 * 50
DIM = 64
NC, NS = 2, 16
NW = NC * NS
BPW = NUM_TOKENS // NW
CHUNK = 128
NCHUNK = BPW // CHUNK
NBUF = 8
LAG = NBUF // 2


def _emb_body(idx_hbm, table_hbm, out_hbm, idx_v, rows, sg, so):
    wid = lax.axis_index("s") * NC + lax.axis_index("c")
    base = wid * BPW
    pltpu.sync_copy(idx_hbm.at[wid], idx_v)

    def gather(c, b):
        pltpu.async_copy(table_hbm.at[idx_v.at[c]], rows[b], sg[b])

    def gather_wait(c, b):
        pltpu.make_async_copy(table_hbm.at[idx_v.at[c]], rows[b], sg[b]).wait()

    def write_out(c, b):
        pltpu.async_copy(rows[b], out_hbm.at[pl.ds(base + c * CHUNK, CHUNK)],
                         so[b])

    def write_wait(c, b):
        pltpu.make_async_copy(
            rows[b], out_hbm.at[pl.ds(base + c * CHUNK, CHUNK)], so[b]).wait()

    for i in range(NBUF):
        gather(i, i)
        if i >= LAG:
            j = i - LAG
            gather_wait(j, j)
            write_out(j, j)

    def group(g, _):
        for b in range(NBUF):
            i = NBUF * g + b
            j = i - LAG
            bj = (b - LAG) % NBUF
            write_wait(i - NBUF, b)
            gather(i, b)
            gather_wait(j, bj)
            write_out(j, bj)
        return _

    lax.fori_loop(1, NCHUNK // NBUF, group, None)

    for j in range(NCHUNK - LAG, NCHUNK):
        bj = j % NBUF
        gather_wait(j, bj)
        write_out(j, bj)
    for j in range(NCHUNK - NBUF, NCHUNK):
        write_wait(j, j % NBUF)


@jax.jit
def _embedding_lookup(idx3, weight):
    mesh = plsc.VectorSubcoreMesh(core_axis_name="c", subcore_axis_name="s")
    k = functools.partial(
        pl.kernel,
        out_type=jax.ShapeDtypeStruct((NUM_TOKENS, DIM), jnp.float32),
        mesh=mesh,
        scratch_types=[
            pltpu.VMEM((NCHUNK, CHUNK), jnp.int32),
            [pltpu.VMEM((CHUNK, DIM), jnp.float32) for _ in range(NBUF)],
            [pltpu.SemaphoreType.DMA for _ in range(NBUF)],
            [pltpu.SemaphoreType.DMA for _ in range(NBUF)],
        ],
        compiler_params=pltpu.CompilerParams(use_tc_tiling_on_sc=False),
    )(_emb_body)
    return k(idx3, weight)


def kernel(token_ids, weight):
    idx3 = token_ids.astype(jnp.int32).reshape(NW, NCHUNK, CHUNK)
    out = _embedding_lookup(idx3, weight)
    return out.reshape(token_ids.shape + (DIM,))
